# baseline (speedup 1.0000x reference)
import functools

import jax
import jax.numpy as jnp
from jax.experimental import pallas as pl
from jax.experimental.pallas import tpu as pltpu

_NUM_EXPERTS = 64
_TOP_K = 8
_TOKEN_BLOCK = 1024


def _moe_block_kernel(hs_ref, gw_ref, out_ref, buf_ref, c_ref):
    i = pl.program_id(0)
    n = pl.num_programs(0) - 1

    @pl.when(i < n)
    def _route():
        hs = hs_ref[...]
        buf_ref[i % 2] = hs
        logits = jax.lax.dot_general(
            hs, gw_ref[...],
            dimension_numbers=(((1,), (1,)), ((), ())),
            preferred_element_type=jnp.float32,
        )
        cur = logits
        m = None
        kth = None
        for _ in range(_TOP_K):
            kth = jnp.max(cur, axis=-1, keepdims=True)
            if m is None:
                m = kth
            cur = jnp.where(cur >= kth, -jnp.inf, cur)
        sel = logits >= kth
        e = jnp.exp(logits - m)
        q = jnp.sum(jnp.where(sel, e, 0.0), axis=-1, keepdims=True)
        norm_sum = q / q
        c_ref[i % 2] = jnp.broadcast_to(1.05 * norm_sum, (_TOKEN_BLOCK, 128))

    @pl.when(i >= 1)
    def _emit():
        prev = (i - 1) % 2
        out_ref[...] = buf_ref[prev] * c_ref[prev][:, 0:1]


@functools.partial(jax.jit, static_argnames=())
def kernel(hidden_states, gate_weight):
    b, s, h = hidden_states.shape
    t = b * s
    hs = hidden_states.reshape(t, h)
    n = t // _TOKEN_BLOCK
    out = pl.pallas_call(
        _moe_block_kernel,
        grid=(n + 1,),
        in_specs=[
            pl.BlockSpec(
                (_TOKEN_BLOCK, h),
                lambda i, _last=n - 1: (jnp.minimum(i, _last), 0),
            ),
            pl.BlockSpec((_NUM_EXPERTS, h), lambda i: (0, 0)),
        ],
        out_specs=pl.BlockSpec(
            (_TOKEN_BLOCK, h), lambda i: (jnp.maximum(i - 1, 0), 0)
        ),
        out_shape=jax.ShapeDtypeStruct((t, h), hidden_states.dtype),
        scratch_shapes=[
            pltpu.VMEM((2, _TOKEN_BLOCK, h), jnp.float32),
            pltpu.VMEM((2, _TOKEN_BLOCK, 128), jnp.float32),
        ],
        compiler_params=pltpu.CompilerParams(
            dimension_semantics=("arbitrary",),
        ),
    )(hs, gate_weight)
    return out.reshape(b, s, h)


# speedup vs baseline: 1.0525x; 1.0525x over previous
import functools

import jax
import jax.numpy as jnp
from jax.experimental import pallas as pl
from jax.experimental.pallas import tpu as pltpu

_NUM_EXPERTS = 64
_TOP_K = 8
_TOKEN_BLOCK = 1024


def _moe_block_kernel(hs_ref, gw_ref, out_ref):
    hs = hs_ref[...]
    logits = jax.lax.dot_general(
        hs, gw_ref[...],
        dimension_numbers=(((1,), (1,)), ((), ())),
        preferred_element_type=jnp.float32,
        precision=jax.lax.Precision.DEFAULT,
    )
    cur = logits
    m = None
    kth = None
    for _ in range(_TOP_K):
        kth = jnp.max(cur, axis=-1, keepdims=True)
        if m is None:
            m = kth
        cur = jnp.where(cur >= kth, -jnp.inf, cur)
    sel = logits >= kth
    e = jnp.exp(logits - m)
    q = jnp.sum(jnp.where(sel, e, 0.0), axis=-1, keepdims=True)
    norm_sum = q / q
    out_ref[...] = hs * (1.05 * norm_sum)


@functools.partial(jax.jit, static_argnames=())
def kernel(hidden_states, gate_weight):
    b, s, h = hidden_states.shape
    t = b * s
    hs = hidden_states.reshape(t, h)
    grid = (t // _TOKEN_BLOCK,)
    out = pl.pallas_call(
        _moe_block_kernel,
        grid=grid,
        in_specs=[
            pl.BlockSpec((_TOKEN_BLOCK, h), lambda i: (i, 0)),
            pl.BlockSpec((_NUM_EXPERTS, h), lambda i: (0, 0)),
        ],
        out_specs=pl.BlockSpec((_TOKEN_BLOCK, h), lambda i: (i, 0)),
        out_shape=jax.ShapeDtypeStruct((t, h), hidden_states.dtype),
        compiler_params=pltpu.CompilerParams(
            dimension_semantics=("parallel",),
        ),
    )(hs, gate_weight)
    return out.reshape(b, s, h)
